# speedup vs baseline: 2.4736x; 2.4736x over previous
import jax
import jax.numpy as jnp
from jax import lax
from jax.experimental import pallas as pl
from jax.experimental.pallas import tpu as pltpu
from jax.experimental.pallas import tpu_sc as plsc

_EMBED = 4096
_B = 1024 * 20
_NC = 2
_NS = 16
_NW = _NC * _NS
_BPW = _B // _NW
_C = 8
_NCHUNK = _BPW // _C


def _emb_body(table_hbm, idx_hbm, out_hbm, idx_v, buf0, buf1, sem0, sem1):
    wid = lax.axis_index("s") * _NC + lax.axis_index("c")
    base = wid * _BPW
    pltpu.sync_copy(idx_hbm.at[pl.ds(base, _BPW)], idx_v)

    def gather_start(chunk, buf, sem):
        pltpu.async_copy(table_hbm.at[idx_v.at[pl.ds(chunk * _C, _C)]], buf, sem)

    def gather_wait(chunk, buf, sem):
        pltpu.make_async_copy(
            table_hbm.at[idx_v.at[pl.ds(chunk * _C, _C)]], buf, sem
        ).wait()

    gather_start(0, buf0, sem0)

    @pl.loop(0, _NCHUNK, step=2)
    def _(g):
        for b in range(2):
            buf, sem = (buf0, sem0) if b == 0 else (buf1, sem1)
            nxt_buf, nxt_sem = (buf1, sem1) if b == 0 else (buf0, sem0)
            c = g + b

            @pl.when(c + 1 < _NCHUNK)
            def _():
                gather_start(c + 1, nxt_buf, nxt_sem)

            gather_wait(c, buf, sem)
            pltpu.sync_copy(buf, out_hbm.at[pl.ds(base + c * _C, _C)])


@jax.jit
def kernel(x, table):
    idx = x.reshape(-1).astype(jnp.int32)
    mesh = plsc.VectorSubcoreMesh(core_axis_name="c", subcore_axis_name="s")
    out = pl.kernel(
        _emb_body,
        out_type=jax.ShapeDtypeStruct((_B, _EMBED), jnp.float32),
        mesh=mesh,
        scratch_types=[
            pltpu.VMEM((_BPW,), jnp.int32),
            pltpu.VMEM((_C, _EMBED), jnp.float32),
            pltpu.VMEM((_C, _EMBED), jnp.float32),
            pltpu.SemaphoreType.DMA,
            pltpu.SemaphoreType.DMA,
        ],
    )(table, idx)
    return out.reshape(x.shape[0], -1)


# speedup vs baseline: 2.4761x; 1.0010x over previous
import jax
import jax.numpy as jnp
from jax import lax
from jax.experimental import pallas as pl
from jax.experimental.pallas import tpu as pltpu
from jax.experimental.pallas import tpu_sc as plsc

_EMBED = 4096
_B = 1024 * 20
_NC = 2
_NS = 16
_NW = _NC * _NS
_BPW = _B // _NW
_C = 8
_NCHUNK = _BPW // _C
_NBUF = 3
_MAIN = _NCHUNK - (_NCHUNK % _NBUF)


def _emb_body(table_hbm, idx_hbm, out_hbm, idx_v,
              buf0, buf1, buf2, g0, g1, g2, o0, o1, o2):
    bufs = (buf0, buf1, buf2)
    gsems = (g0, g1, g2)
    osems = (o0, o1, o2)

    wid = lax.axis_index("s") * _NC + lax.axis_index("c")
    base = wid * _BPW
    pltpu.sync_copy(idx_hbm.at[pl.ds(base, _BPW)], idx_v)

    def gather_start(chunk, b):
        pltpu.async_copy(
            table_hbm.at[idx_v.at[pl.ds(chunk * _C, _C)]], bufs[b], gsems[b])

    def gather_wait(chunk, b):
        pltpu.make_async_copy(
            table_hbm.at[idx_v.at[pl.ds(chunk * _C, _C)]], bufs[b], gsems[b]
        ).wait()

    def out_start(chunk, b):
        pltpu.async_copy(
            bufs[b], out_hbm.at[pl.ds(base + chunk * _C, _C)], osems[b])

    def out_wait(chunk, b):
        pltpu.make_async_copy(
            bufs[b], out_hbm.at[pl.ds(base + chunk * _C, _C)], osems[b]
        ).wait()

    gather_start(0, 0)
    gather_start(1, 1)

    def step(c, b):
        gather_wait(c, b)
        out_start(c, b)
        b2 = (b + 2) % _NBUF

        @pl.when(c + 2 < _NCHUNK)
        def _():
            @pl.when(c >= 1)
            def _():
                out_wait(c - 1, b2)

            gather_start(c + 2, b2)

    @pl.loop(0, _MAIN, step=_NBUF)
    def _(g):
        for b in range(_NBUF):
            step(g + b, b)

    for c in range(_MAIN, _NCHUNK):
        step(c, c % _NBUF)

    for c in range(_NCHUNK - _NBUF, _NCHUNK):
        out_wait(c, c % _NBUF)


@jax.jit
def kernel(x, table):
    idx = x.reshape(-1).astype(jnp.int32)
    mesh = plsc.VectorSubcoreMesh(core_axis_name="c", subcore_axis_name="s")
    out = pl.kernel(
        _emb_body,
        out_type=jax.ShapeDtypeStruct((_B, _EMBED), jnp.float32),
        mesh=mesh,
        scratch_types=[
            pltpu.VMEM((_BPW,), jnp.int32),
            pltpu.VMEM((_C, _EMBED), jnp.float32),
            pltpu.VMEM((_C, _EMBED), jnp.float32),
            pltpu.VMEM((_C, _EMBED), jnp.float32),
            pltpu.SemaphoreType.DMA,
            pltpu.SemaphoreType.DMA,
            pltpu.SemaphoreType.DMA,
            pltpu.SemaphoreType.DMA,
            pltpu.SemaphoreType.DMA,
            pltpu.SemaphoreType.DMA,
        ],
    )(table, idx)
    return out.reshape(x.shape[0], -1)


# speedup vs baseline: 5.0390x; 2.0351x over previous
import jax
import jax.numpy as jnp
from jax import lax
from jax.experimental import pallas as pl
from jax.experimental.pallas import tpu as pltpu
from jax.experimental.pallas import tpu_sc as plsc

_EMBED = 4096
_ROWS = 1024
_L = 20
_NC = 2
_NS = 16
_NW = _NC * _NS
_RPW = _ROWS // _NW
_C = 8
_Q = _RPW // _C
_NCHUNK = _L * _Q
_NBUF = 3
_MAIN = _NCHUNK - (_NCHUNK % _NBUF)


def _emb_body(table_hbm, idx_hbm, out_hbm, idx_v,
              buf0, buf1, buf2, g0, g1, g2, o0, o1, o2):
    bufs = (buf0, buf1, buf2)
    gsems = (g0, g1, g2)
    osems = (o0, o1, o2)

    wid = lax.axis_index("s") * _NC + lax.axis_index("c")
    row_base = wid * _RPW
    pltpu.sync_copy(idx_hbm.at[wid], idx_v)

    def out_slice(c):
        i0 = row_base + (c % _Q) * _C
        col0 = (c // _Q) * _EMBED
        return out_hbm.at[pl.ds(i0, _C), pl.ds(col0, _EMBED)]

    def gather_start(c, b):
        pltpu.async_copy(table_hbm.at[idx_v.at[c]], bufs[b], gsems[b])

    def gather_wait(c, b):
        pltpu.make_async_copy(table_hbm.at[idx_v.at[c]], bufs[b], gsems[b]).wait()

    def out_start(c, b):
        pltpu.async_copy(bufs[b], out_slice(c), osems[b])

    def out_wait(c, b):
        pltpu.make_async_copy(bufs[b], out_slice(c), osems[b]).wait()

    gather_start(0, 0)
    gather_start(1, 1)

    def step(c, b):
        gather_wait(c, b)
        out_start(c, b)
        b2 = (b + 2) % _NBUF

        @pl.when(c + 2 < _NCHUNK)
        def _():
            @pl.when(c >= 1)
            def _():
                out_wait(c - 1, b2)

            gather_start(c + 2, b2)

    @pl.loop(0, _MAIN, step=_NBUF)
    def _(g):
        for b in range(_NBUF):
            step(g + b, b)

    for c in range(_MAIN, _NCHUNK):
        step(c, c % _NBUF)

    for c in range(_NCHUNK - _NBUF, _NCHUNK):
        out_wait(c, c % _NBUF)


@jax.jit
def kernel(x, table):
    idxarr = (
        x.astype(jnp.int32)
        .reshape(_NW, _Q, _C, _L)
        .transpose(0, 3, 1, 2)
        .reshape(_NW, _NCHUNK, _C)
    )
    mesh = plsc.VectorSubcoreMesh(core_axis_name="c", subcore_axis_name="s")
    return pl.kernel(
        _emb_body,
        out_type=jax.ShapeDtypeStruct((_ROWS, _L * _EMBED), jnp.float32),
        mesh=mesh,
        scratch_types=[
            pltpu.VMEM((_NCHUNK, _C), jnp.int32),
            pltpu.VMEM((_C, _EMBED), jnp.float32),
            pltpu.VMEM((_C, _EMBED), jnp.float32),
            pltpu.VMEM((_C, _EMBED), jnp.float32),
            pltpu.SemaphoreType.DMA,
            pltpu.SemaphoreType.DMA,
            pltpu.SemaphoreType.DMA,
            pltpu.SemaphoreType.DMA,
            pltpu.SemaphoreType.DMA,
            pltpu.SemaphoreType.DMA,
        ],
    )(table, idxarr)
